# baseline (speedup 1.0000x reference)
import jax
import jax.numpy as jnp
from jax import lax
from jax.experimental import pallas as pl
from jax.experimental.pallas import tpu as pltpu
from jax.experimental.pallas import tpu_sc as plsc

V = 50000
D = 128
HH = 128
G4 = 4 * HH
B = 8
L = 512
K = 10
START = 8
STOP = 9

NW = 32
TOT = 2 * B * L
ROWS_PER_W = TOT // NW
IDX_ROWS = TOT // 128


def _sc_gather_body(table, idx, out, idx_v, rows_v, sem):
    wid = lax.axis_index("s") * 2 + lax.axis_index("c")
    pltpu.sync_copy(idx.at[pl.ds(wid * 2, 2)], idx_v)
    pltpu.async_copy(table.at[idx_v.at[0]], rows_v.at[pl.ds(0, 128)], sem).wait()
    pltpu.async_copy(table.at[idx_v.at[1]], rows_v.at[pl.ds(128, 128)], sem).wait()
    pltpu.sync_copy(rows_v, out.at[pl.ds(wid * ROWS_PER_W, ROWS_PER_W)])


def _sc_gather(table, idx2d):
    call = pl.kernel(
        _sc_gather_body,
        out_type=jax.ShapeDtypeStruct((TOT, D), jnp.float32),
        mesh=plsc.VectorSubcoreMesh(core_axis_name="c", subcore_axis_name="s"),
        scratch_types=[
            pltpu.VMEM((2, 128), jnp.int32),
            pltpu.VMEM((ROWS_PER_W, D), jnp.float32),
            pltpu.SemaphoreType.DMA,
        ],
    )
    return call(table, idx2d)


def _tc_body(emb_ref, wihf_ref, wihb_ref, whhf_ref, whhb_ref,
             bihf_ref, bhhf_ref, bihb_ref, bhhb_ref,
             wout_ref, bout_ref, trans_ref, len16_ref,
             len8T_ref, lens_ref, h0_ref, c0_ref,
             scores_ref, paths_ref,
             pf_ref, pb_ref, outf_ref, outb_ref, feats_ref, bptrs_ref,
             pathsT_ref):
    dnT = (((1,), (1,)), ((), ()))
    biasf = (bihf_ref[...] + bhhf_ref[...]).reshape(1, G4)
    biasb = (bihb_ref[...] + bhhb_ref[...]).reshape(1, G4)
    wihf = wihf_ref[...]
    wihb = wihb_ref[...]
    for cchunk in range(4):
        r0 = cchunk * 1024
        t0 = cchunk * 128
        pf = lax.dot_general(emb_ref[r0:r0 + 1024, :], wihf, dnT,
                             preferred_element_type=jnp.float32) + biasf
        pf_ref[t0:t0 + 128] = pf.reshape(128, B, G4)
        pb = lax.dot_general(emb_ref[B * L + r0:B * L + r0 + 1024, :], wihb,
                             dnT, preferred_element_type=jnp.float32) + biasb
        pb_ref[t0:t0 + 128] = pb.reshape(128, B, G4)

    whf = whhf_ref[...]
    whb = whhb_ref[...]
    len16 = len16_ref[...]
    lens = [lens_ref[b] for b in range(B)]

    def lstm_step(s, carry):
        h, c = carry
        gf = pf_ref[s] + lax.dot_general(h[0:B], whf, dnT,
                                         preferred_element_type=jnp.float32)
        gb = pb_ref[s] + lax.dot_general(h[B:2 * B], whb, dnT,
                                         preferred_element_type=jnp.float32)
        g = jnp.concatenate([gf, gb], axis=0)
        i_g = jax.nn.sigmoid(g[:, 0:HH])
        f_g = jax.nn.sigmoid(g[:, HH:2 * HH])
        g_g = jnp.tanh(g[:, 2 * HH:3 * HH])
        o_g = jax.nn.sigmoid(g[:, 3 * HH:4 * HH])
        c_new = f_g * c + i_g * g_g
        h_new = o_g * jnp.tanh(c_new)
        m = s < len16
        h2 = jnp.where(m, h_new, h)
        c2 = jnp.where(m, c_new, c)
        outrow = jnp.where(m, h_new, 0.0)
        outf_ref[pl.ds(s, 1)] = outrow[0:B][None]
        for b in range(B):
            lb = lens[b]
            tgt = jnp.where(s < lb, lb - 1 - s, s)
            outb_ref[pl.ds(tgt, 1), pl.ds(b, 1)] = outrow[B + b:B + b + 1, :][None]
        return h2, c2

    h0c = jnp.concatenate([h0_ref[0], h0_ref[1]], axis=0)
    c0c = jnp.concatenate([c0_ref[0], c0_ref[1]], axis=0)
    lax.fori_loop(0, L, lstm_step, (h0c, c0c), unroll=4)

    wof = wout_ref[:, 0:HH]
    wob = wout_ref[:, HH:2 * HH]
    bout = bout_ref[...].reshape(1, K)
    for cchunk in range(4):
        t0 = cchunk * 128
        of = outf_ref[pl.ds(t0, 128)].reshape(1024, HH)
        ob = outb_ref[pl.ds(t0, 128)].reshape(1024, HH)
        fe = (lax.dot_general(of, wof, dnT,
                              preferred_element_type=jnp.float32) +
              lax.dot_general(ob, wob, dnT,
                              preferred_element_type=jnp.float32)) + bout
        feats_ref[t0:t0 + 128] = jnp.swapaxes(fe.reshape(128, B, K), 1, 2)

    trans = trans_ref[...]
    len8T = len8T_ref[...]
    iota_kT = lax.broadcasted_iota(jnp.int32, (K, B), 0)
    fv0T = jnp.where(iota_kT == START, 0.0, -10000.0)
    tcols = [jnp.broadcast_to(trans_ref[:, p:p + 1], (K, B)) for p in range(K)]
    idx_consts = [jnp.full((K, B), p, jnp.int32) for p in range(K)]

    def vstep(t, fvT):
        vals = [jnp.broadcast_to(fvT[p:p + 1, :], (K, B)) + tcols[p]
                for p in range(K)]
        idxs = list(idx_consts)
        while len(vals) > 1:
            nv, ni = [], []
            for a in range(0, len(vals) - 1, 2):
                cond = vals[a] >= vals[a + 1]
                nv.append(jnp.where(cond, vals[a], vals[a + 1]))
                ni.append(jnp.where(cond, idxs[a], idxs[a + 1]))
            if len(vals) % 2:
                nv.append(vals[-1])
                ni.append(idxs[-1])
            vals, idxs = nv, ni
        vmaxT, bptrT = vals[0], idxs[0]
        bptrs_ref[pl.ds(t, 1)] = bptrT[None]
        fv_new = vmaxT + feats_ref[t]
        return jnp.where(t < len8T, fv_new, fvT)

    fvT = lax.fori_loop(0, L, vstep, fv0T, unroll=2)
    fv = jnp.swapaxes(fvT, 0, 1)
    iota_k = lax.broadcasted_iota(jnp.int32, (B, K), 1)
    terminal = fv + trans[STOP:STOP + 1, :]
    smax = jnp.max(terminal, axis=1, keepdims=True)
    best = jnp.min(jnp.where(terminal == smax, iota_k, K), axis=1,
                   keepdims=True).astype(jnp.int32)
    scores_ref[...] = smax

    bestT = jnp.reshape(best, (1, B))

    def btstep(r, curT):
        t = L - 1 - r
        bptrT = bptrs_ref[t]
        curT = jnp.where(t == len8T - 1, bestT, curT)
        active = t < len8T
        pathsT_ref[pl.ds(t, 1)] = jnp.where(active, curT, -1)[None]
        prevT = jnp.max(jnp.where(iota_kT == curT, bptrT, 0), axis=0,
                        keepdims=True)
        return jnp.where(active, prevT, curT)

    lax.fori_loop(0, L, btstep, jnp.zeros((1, B), jnp.int32), unroll=2)
    for c in range(4):
        chunk = pathsT_ref[pl.ds(c * 128, 128)]
        paths_ref[:, c * 128:(c + 1) * 128] = \
            jnp.swapaxes(chunk, 0, 2).reshape(B, 128)


_TC_OUT_SHAPES = [jax.ShapeDtypeStruct((B, 1), jnp.float32),
                  jax.ShapeDtypeStruct((B, L), jnp.int32)]
_TC_IN_SPECS = [pl.BlockSpec(memory_space=pltpu.VMEM)] * 14 + \
               [pl.BlockSpec(memory_space=pltpu.SMEM)] + \
               [pl.BlockSpec(memory_space=pltpu.VMEM)] * 2
_TC_SCRATCH = [
    pltpu.VMEM((L, B, G4), jnp.float32),
    pltpu.VMEM((L, B, G4), jnp.float32),
    pltpu.VMEM((L, B, HH), jnp.float32),
    pltpu.VMEM((L, B, HH), jnp.float32),
    pltpu.VMEM((L, K, B), jnp.float32),
    pltpu.VMEM((L, K, B), jnp.int32),
    pltpu.VMEM((L, 1, B), jnp.int32),
]

_tc_call = pl.pallas_call(
    _tc_body,
    out_shape=_TC_OUT_SHAPES,
    in_specs=_TC_IN_SPECS,
    out_specs=[pl.BlockSpec(memory_space=pltpu.VMEM)] * 2,
    scratch_shapes=_TC_SCRATCH,
)


def _prep_tc_inputs(sentences, lengths, Wih_f, Whh_f, bih_f, bhh_f,
                    Wih_b, Whh_b, bih_b, bhh_b, W_out, b_out, transitions,
                    h0, c0):
    lengths = lengths.astype(jnp.int32)
    len16 = jnp.concatenate([lengths, lengths])[:, None]
    len8T = lengths[None, :]
    return (Wih_f, Wih_b, Whh_f, Whh_b, bih_f, bhh_f, bih_b, bhh_b,
            W_out, b_out, transitions, len16, len8T, lengths, h0, c0)


def _gather_indices(sentences, lengths):
    lengths = lengths.astype(jnp.int32)
    t_ar = jnp.arange(L, dtype=jnp.int32)[None]
    idx_rev = jnp.where(t_ar < lengths[:, None], lengths[:, None] - 1 - t_ar,
                        t_ar)
    sent_rev = jnp.take_along_axis(sentences, idx_rev, axis=1)
    return jnp.concatenate([sentences.T.reshape(-1),
                            sent_rev.T.reshape(-1)]).reshape(IDX_ROWS, 128)


def kernel(sentences, lengths, W_emb, Wih_f, Whh_f, bih_f, bhh_f,
           Wih_b, Whh_b, bih_b, bhh_b, W_out, b_out, transitions, h0, c0):
    idx_all = _gather_indices(sentences, lengths)
    emb = _sc_gather(W_emb, idx_all)
    tc_in = _prep_tc_inputs(sentences, lengths, Wih_f, Whh_f, bih_f, bhh_f,
                            Wih_b, Whh_b, bih_b, bhh_b, W_out, b_out,
                            transitions, h0, c0)
    scores2, paths = _tc_call(emb, *tc_in)
    return scores2[:, 0], paths


# speedup vs baseline: 1.0417x; 1.0417x over previous
import jax
import jax.numpy as jnp
from jax import lax
from jax.experimental import pallas as pl
from jax.experimental.pallas import tpu as pltpu
from jax.experimental.pallas import tpu_sc as plsc

V = 50000
D = 128
HH = 128
G4 = 4 * HH
B = 8
L = 512
K = 10
START = 8
STOP = 9

NW = 32
TOT = 2 * B * L
ROWS_PER_W = TOT // NW
IDX_ROWS = TOT // 128


def _sc_gather_body(table, idx, out, idx_v, rows_v, sem):
    wid = lax.axis_index("s") * 2 + lax.axis_index("c")
    pltpu.sync_copy(idx.at[pl.ds(wid * 2, 2)], idx_v)
    pltpu.async_copy(table.at[idx_v.at[0]], rows_v.at[pl.ds(0, 128)], sem).wait()
    pltpu.async_copy(table.at[idx_v.at[1]], rows_v.at[pl.ds(128, 128)], sem).wait()
    pltpu.sync_copy(rows_v, out.at[pl.ds(wid * ROWS_PER_W, ROWS_PER_W)])


def _sc_gather(table, idx2d):
    call = pl.kernel(
        _sc_gather_body,
        out_type=jax.ShapeDtypeStruct((TOT, D), jnp.float32),
        mesh=plsc.VectorSubcoreMesh(core_axis_name="c", subcore_axis_name="s"),
        scratch_types=[
            pltpu.VMEM((2, 128), jnp.int32),
            pltpu.VMEM((ROWS_PER_W, D), jnp.float32),
            pltpu.SemaphoreType.DMA,
        ],
    )
    return call(table, idx2d)


def _tc_body(emb_ref, wihf_ref, wihb_ref, whhfT_ref, whhbT_ref,
             bihf_ref, bhhf_ref, bihb_ref, bhhb_ref,
             wout_ref, bout_ref, trans_ref, len16_ref,
             len8T_ref, lens_ref, h0_ref, c0_ref,
             scores_ref, paths_ref,
             pf_ref, pb_ref, outf_ref, outb_ref, feats_ref, bptrs_ref,
             pathsT_ref):
    dnT = (((1,), (1,)), ((), ()))
    biasf = (bihf_ref[...] + bhhf_ref[...]).reshape(1, G4)
    biasb = (bihb_ref[...] + bhhb_ref[...]).reshape(1, G4)
    wihf = wihf_ref[...]
    wihb = wihb_ref[...]
    for cchunk in range(4):
        r0 = cchunk * 1024
        t0 = cchunk * 128
        pf = lax.dot_general(emb_ref[r0:r0 + 1024, :], wihf, dnT,
                             preferred_element_type=jnp.float32) + biasf
        pf_ref[t0:t0 + 128] = pf.reshape(128, B, G4)
        pb = lax.dot_general(emb_ref[B * L + r0:B * L + r0 + 1024, :], wihb,
                             dnT, preferred_element_type=jnp.float32) + biasb
        pb_ref[t0:t0 + 128] = pb.reshape(128, B, G4)

    whf = whhfT_ref[...]
    whb = whhbT_ref[...]
    len16 = len16_ref[...]
    lens = [lens_ref[b] for b in range(B)]

    def lstm_step(s, carry):
        h, c = carry
        gf = pf_ref[s] + jnp.dot(h[0:B], whf,
                                 preferred_element_type=jnp.float32)
        gb = pb_ref[s] + jnp.dot(h[B:2 * B], whb,
                                 preferred_element_type=jnp.float32)
        g = jnp.concatenate([gf, gb], axis=0)
        i_g = jax.nn.sigmoid(g[:, 0:HH])
        f_g = jax.nn.sigmoid(g[:, HH:2 * HH])
        g_g = jnp.tanh(g[:, 2 * HH:3 * HH])
        o_g = jax.nn.sigmoid(g[:, 3 * HH:4 * HH])
        c_new = f_g * c + i_g * g_g
        h_new = o_g * jnp.tanh(c_new)
        m = s < len16
        h2 = jnp.where(m, h_new, h)
        c2 = jnp.where(m, c_new, c)
        outrow = jnp.where(m, h_new, 0.0)
        outf_ref[pl.ds(s, 1)] = outrow[0:B][None]
        for b in range(B):
            lb = lens[b]
            tgt = jnp.where(s < lb, lb - 1 - s, s)
            outb_ref[pl.ds(tgt, 1), pl.ds(b, 1)] = outrow[B + b:B + b + 1, :][None]
        return h2, c2

    h0c = jnp.concatenate([h0_ref[0], h0_ref[1]], axis=0)
    c0c = jnp.concatenate([c0_ref[0], c0_ref[1]], axis=0)
    lax.fori_loop(0, L, lstm_step, (h0c, c0c), unroll=4)

    wof = wout_ref[:, 0:HH]
    wob = wout_ref[:, HH:2 * HH]
    bout = bout_ref[...].reshape(1, K)
    for cchunk in range(4):
        t0 = cchunk * 128
        of = outf_ref[pl.ds(t0, 128)].reshape(1024, HH)
        ob = outb_ref[pl.ds(t0, 128)].reshape(1024, HH)
        fe = (lax.dot_general(of, wof, dnT,
                              preferred_element_type=jnp.float32) +
              lax.dot_general(ob, wob, dnT,
                              preferred_element_type=jnp.float32)) + bout
        feats_ref[t0:t0 + 128] = jnp.swapaxes(fe.reshape(128, B, K), 1, 2)

    trans = trans_ref[...]
    len8T = len8T_ref[...]
    iota_kT = lax.broadcasted_iota(jnp.int32, (K, B), 0)
    fv0T = jnp.where(iota_kT == START, 0.0, -10000.0)
    tcols = [jnp.broadcast_to(trans_ref[:, p:p + 1], (K, B)) for p in range(K)]
    idx_consts = [jnp.full((K, B), p, jnp.int32) for p in range(K)]

    def vstep(t, fvT):
        vals = [jnp.broadcast_to(fvT[p:p + 1, :], (K, B)) + tcols[p]
                for p in range(K)]
        idxs = list(idx_consts)
        while len(vals) > 1:
            nv, ni = [], []
            for a in range(0, len(vals) - 1, 2):
                cond = vals[a] >= vals[a + 1]
                nv.append(jnp.where(cond, vals[a], vals[a + 1]))
                ni.append(jnp.where(cond, idxs[a], idxs[a + 1]))
            if len(vals) % 2:
                nv.append(vals[-1])
                ni.append(idxs[-1])
            vals, idxs = nv, ni
        vmaxT, bptrT = vals[0], idxs[0]
        bptrs_ref[pl.ds(t, 1)] = bptrT[None]
        fv_new = vmaxT + feats_ref[t]
        return jnp.where(t < len8T, fv_new, fvT)

    fvT = lax.fori_loop(0, L, vstep, fv0T, unroll=2)
    fv = jnp.swapaxes(fvT, 0, 1)
    iota_k = lax.broadcasted_iota(jnp.int32, (B, K), 1)
    terminal = fv + trans[STOP:STOP + 1, :]
    smax = jnp.max(terminal, axis=1, keepdims=True)
    best = jnp.min(jnp.where(terminal == smax, iota_k, K), axis=1,
                   keepdims=True).astype(jnp.int32)
    scores_ref[...] = smax

    bestT = jnp.reshape(best, (1, B))

    def btstep(r, curT):
        t = L - 1 - r
        bptrT = bptrs_ref[t]
        curT = jnp.where(t == len8T - 1, bestT, curT)
        active = t < len8T
        pathsT_ref[pl.ds(t, 1)] = jnp.where(active, curT, -1)[None]
        prevT = jnp.max(jnp.where(iota_kT == curT, bptrT, 0), axis=0,
                        keepdims=True)
        return jnp.where(active, prevT, curT)

    lax.fori_loop(0, L, btstep, jnp.zeros((1, B), jnp.int32), unroll=2)
    for c in range(4):
        chunk = pathsT_ref[pl.ds(c * 128, 128)]
        paths_ref[:, c * 128:(c + 1) * 128] = \
            jnp.swapaxes(chunk, 0, 2).reshape(B, 128)


_TC_OUT_SHAPES = [jax.ShapeDtypeStruct((B, 1), jnp.float32),
                  jax.ShapeDtypeStruct((B, L), jnp.int32)]
_TC_IN_SPECS = [pl.BlockSpec(memory_space=pltpu.VMEM)] * 14 + \
               [pl.BlockSpec(memory_space=pltpu.SMEM)] + \
               [pl.BlockSpec(memory_space=pltpu.VMEM)] * 2
_TC_SCRATCH = [
    pltpu.VMEM((L, B, G4), jnp.float32),
    pltpu.VMEM((L, B, G4), jnp.float32),
    pltpu.VMEM((L, B, HH), jnp.float32),
    pltpu.VMEM((L, B, HH), jnp.float32),
    pltpu.VMEM((L, K, B), jnp.float32),
    pltpu.VMEM((L, K, B), jnp.int32),
    pltpu.VMEM((L, 1, B), jnp.int32),
]

_tc_call = pl.pallas_call(
    _tc_body,
    out_shape=_TC_OUT_SHAPES,
    in_specs=_TC_IN_SPECS,
    out_specs=[pl.BlockSpec(memory_space=pltpu.VMEM)] * 2,
    scratch_shapes=_TC_SCRATCH,
)


def _prep_tc_inputs(sentences, lengths, Wih_f, Whh_f, bih_f, bhh_f,
                    Wih_b, Whh_b, bih_b, bhh_b, W_out, b_out, transitions,
                    h0, c0):
    lengths = lengths.astype(jnp.int32)
    len16 = jnp.concatenate([lengths, lengths])[:, None]
    len8T = lengths[None, :]
    return (Wih_f, Wih_b, Whh_f.T, Whh_b.T, bih_f, bhh_f, bih_b, bhh_b,
            W_out, b_out, transitions, len16, len8T, lengths, h0, c0)


def _gather_indices(sentences, lengths):
    lengths = lengths.astype(jnp.int32)
    t_ar = jnp.arange(L, dtype=jnp.int32)[None]
    idx_rev = jnp.where(t_ar < lengths[:, None], lengths[:, None] - 1 - t_ar,
                        t_ar)
    sent_rev = jnp.take_along_axis(sentences, idx_rev, axis=1)
    return jnp.concatenate([sentences.T.reshape(-1),
                            sent_rev.T.reshape(-1)]).reshape(IDX_ROWS, 128)


def kernel(sentences, lengths, W_emb, Wih_f, Whh_f, bih_f, bhh_f,
           Wih_b, Whh_b, bih_b, bhh_b, W_out, b_out, transitions, h0, c0):
    idx_all = _gather_indices(sentences, lengths)
    emb = _sc_gather(W_emb, idx_all)
    tc_in = _prep_tc_inputs(sentences, lengths, Wih_f, Whh_f, bih_f, bhh_f,
                            Wih_b, Whh_b, bih_b, bhh_b, W_out, b_out,
                            transitions, h0, c0)
    scores2, paths = _tc_call(emb, *tc_in)
    return scores2[:, 0], paths
